# baseline (speedup 1.0000x reference)
import functools

import jax
import jax.numpy as jnp
from jax import lax
from jax.experimental import pallas as pl
from jax.experimental.pallas import tpu as pltpu
from jax.experimental.pallas import tpu_sc as plsc

_D = 32
_NW = 32
_C = 512


@functools.cache
def _make_gather(batch: int, nf: int):
    n_rows = batch * nf
    assert batch % 128 == 0 and n_rows % (_NW * _C) == 0
    bblk = batch // 128
    cpf = batch // _C
    nchunk = n_rows // (_NW * _C)
    b_per_w = n_rows // _NW
    bspan = _C // 128
    mesh = plsc.VectorSubcoreMesh(core_axis_name="c", subcore_axis_name="s")

    @functools.partial(
        pl.kernel,
        mesh=mesh,
        out_type=jax.ShapeDtypeStruct((nf, 4, bblk, 8, 128), jnp.float32),
        compiler_params=pltpu.CompilerParams(use_tc_tiling_on_sc=False, needs_layout_passes=False),
        scratch_types=[
            pltpu.VMEM((b_per_w,), jnp.int32),
            pltpu.VMEM((_C, _D), jnp.float32),
            pltpu.VMEM((_C, _D), jnp.float32),
            pltpu.VMEM((2, 4, bspan, 8, 128), jnp.float32),
            pltpu.SemaphoreType.DMA,
            pltpu.SemaphoreType.DMA,
            pltpu.SemaphoreType.DMA,
            pltpu.SemaphoreType.DMA,
        ],
    )
    def gather(idx_hbm, table_hbm, out_hbm, idx_v, rv0, rv1, rows_t,
               g0, g1, s0, s1):
        rows_v = (rv0, rv1)
        wid = lax.axis_index("s") * 2 + lax.axis_index("c")
        c0 = wid * nchunk
        gsem = (g0, g1)
        ssem = (s0, s1)
        pltpu.sync_copy(idx_hbm.at[pl.ds(c0 * _C, b_per_w)], idx_v)

        iota16 = lax.broadcasted_iota(jnp.int32, (16,), 0)
        cols = [jnp.full((16,), d, dtype=jnp.int32) for d in range(_D)]

        def start_gather(j, buf):
            return pltpu.async_copy(
                table_hbm.at[idx_v.at[pl.ds(j * _C, _C)]],
                rows_v[buf], gsem[buf])

        def start_store(j, buf):
            cg = c0 + j
            f = cg // cpf
            blk0 = (cg % cpf) * bspan
            return pltpu.async_copy(
                rows_t.at[buf],
                out_hbm.at[f, :, pl.ds(blk0, bspan)],
                ssem[buf])

        def transpose(buf):
            def body(g, _):
                row16 = g * 16 + iota16
                blk = g // 8
                bb = (g % 8) * 16
                for dgrp in range(4):
                    for din in range(8):
                        vals = plsc.load_gather(
                            rows_v[buf],
                            [row16, cols[dgrp * 8 + din]])
                        rows_t[buf, dgrp, blk, din, pl.ds(bb, 16)] = vals
                return ()
            lax.fori_loop(0, _C // 16, body, ())

        g = [None, None]
        s = [None, None]
        g[0] = start_gather(0, 0)
        g[1] = start_gather(1, 1)
        for j in range(nchunk):
            buf = j & 1
            g[buf].wait()
            if s[buf] is not None:
                s[buf].wait()
            transpose(buf)
            s[buf] = start_store(j, buf)
            if j + 2 < nchunk:
                g[buf] = start_gather(j + 2, buf)
        s[0].wait()
        s[1].wait()

    return gather


def kernel(x, table):
    b, f = x.shape
    idx = x.T.reshape(-1).astype(jnp.int32)
    out5 = _make_gather(b, f)(idx, table)
    return out5.transpose(2, 4, 0, 1, 3).reshape(b, f, _D)


# speedup vs baseline: 1.0018x; 1.0018x over previous
import functools

import jax
import jax.numpy as jnp
from jax import lax
from jax.experimental import pallas as pl
from jax.experimental.pallas import tpu as pltpu
from jax.experimental.pallas import tpu_sc as plsc

_D = 32
_DP = 128
_NW = 32
_CHUNK = 416


@functools.cache
def _make_gather(n_rows: int):
    assert n_rows % (_NW * _CHUNK) == 0
    b_per_w = n_rows // _NW
    nchunk = b_per_w // _CHUNK
    mesh = plsc.VectorSubcoreMesh(core_axis_name="c", subcore_axis_name="s")

    @functools.partial(
        pl.kernel,
        mesh=mesh,
        out_type=jax.ShapeDtypeStruct((n_rows, _D), jnp.float32),
        compiler_params=pltpu.CompilerParams(use_tc_tiling_on_sc=False),
        scratch_types=[
            pltpu.VMEM((b_per_w,), jnp.int32),
            pltpu.VMEM((2, _CHUNK, _DP), jnp.float32),
            pltpu.SemaphoreType.DMA,
            pltpu.SemaphoreType.DMA,
            pltpu.SemaphoreType.DMA,
            pltpu.SemaphoreType.DMA,
        ],
    )
    def gather(idx_hbm, table_hbm, out_hbm, idx_v, rows_v, g0, g1, s0, s1):
        wid = lax.axis_index("s") * 2 + lax.axis_index("c")
        base = wid * b_per_w
        gsem = (g0, g1)
        ssem = (s0, s1)
        pltpu.sync_copy(idx_hbm.at[pl.ds(base, b_per_w)], idx_v)

        def start_gather(c, buf):
            return pltpu.async_copy(
                table_hbm.at[idx_v.at[pl.ds(c * _CHUNK, _CHUNK)]],
                rows_v.at[buf], gsem[buf])

        def start_store(c, buf):
            return pltpu.async_copy(
                rows_v.at[buf, :, pl.ds(0, _D)],
                out_hbm.at[pl.ds(base + c * _CHUNK, _CHUNK)],
                ssem[buf])

        g = [None, None]
        s = [None, None]
        g[0] = start_gather(0, 0)
        for c in range(nchunk):
            buf = c & 1
            if c + 1 < nchunk:
                if s[1 - buf] is not None:
                    s[1 - buf].wait()
                g[1 - buf] = start_gather(c + 1, 1 - buf)
            g[buf].wait()
            s[buf] = start_store(c, buf)
        s[(nchunk - 2) & 1].wait()
        s[(nchunk - 1) & 1].wait()

    return gather


def kernel(x, table):
    b, f = x.shape
    n = b * f
    idx = x.reshape(-1).astype(jnp.int32)
    tp = jnp.pad(table, ((0, 0), (0, _DP - _D)))
    out = _make_gather(n)(idx, tp)
    return out.reshape(b, f, _D)


# speedup vs baseline: 1.0644x; 1.0625x over previous
import functools

import jax
import jax.numpy as jnp
from jax import lax
from jax.experimental import pallas as pl
from jax.experimental.pallas import tpu as pltpu
from jax.experimental.pallas import tpu_sc as plsc

_D = 32
_NW = 32
_CHUNK = 1664


@functools.cache
def _make_gather(n_rows: int):
    assert n_rows % (_NW * _CHUNK) == 0
    b_per_w = n_rows // _NW
    nchunk = b_per_w // _CHUNK
    mesh = plsc.VectorSubcoreMesh(core_axis_name="c", subcore_axis_name="s")

    @functools.partial(
        pl.kernel,
        mesh=mesh,
        out_type=jax.ShapeDtypeStruct((n_rows, _D), jnp.float32),
        compiler_params=pltpu.CompilerParams(use_tc_tiling_on_sc=False),
        scratch_types=[
            pltpu.VMEM((b_per_w,), jnp.int32),
            pltpu.VMEM((2, _CHUNK, _D), jnp.float32),
            pltpu.SemaphoreType.DMA,
            pltpu.SemaphoreType.DMA,
            pltpu.SemaphoreType.DMA,
            pltpu.SemaphoreType.DMA,
        ],
    )
    def gather(idx_hbm, table_hbm, out_hbm, idx_v, rows_v, g0, g1, s0, s1):
        wid = lax.axis_index("s") * 2 + lax.axis_index("c")
        base = wid * b_per_w
        gsem = (g0, g1)
        ssem = (s0, s1)
        pltpu.sync_copy(idx_hbm.at[pl.ds(base, b_per_w)], idx_v)

        def start_gather(c, buf):
            return pltpu.async_copy(
                table_hbm.at[idx_v.at[pl.ds(c * _CHUNK, _CHUNK)]],
                rows_v.at[buf], gsem[buf])

        def start_store(c, buf):
            return pltpu.async_copy(
                rows_v.at[buf],
                out_hbm.at[pl.ds(base + c * _CHUNK, _CHUNK)],
                ssem[buf])

        g = [None, None]
        s = [None, None]
        g[0] = start_gather(0, 0)
        for c in range(nchunk):
            buf = c & 1
            if c + 1 < nchunk:
                if s[1 - buf] is not None:
                    s[1 - buf].wait()
                g[1 - buf] = start_gather(c + 1, 1 - buf)
            g[buf].wait()
            s[buf] = start_store(c, buf)
        s[(nchunk - 2) & 1].wait()
        s[(nchunk - 1) & 1].wait()

    return gather


def kernel(x, table):
    b, f = x.shape
    n = b * f
    idx = x.reshape(-1).astype(jnp.int32)
    out = _make_gather(n)(idx, table)
    return out.reshape(b, f, _D)
